# baseline (speedup 1.0000x reference)
import functools

import jax
import jax.numpy as jnp
from jax.experimental import pallas as pl
from jax.experimental.pallas import tpu as pltpu

B, N, F, H, O, R = 32, 512, 256, 128, 128, 53
G = 4


def _gcn_kernel(x_hbm, adj_hbm, W1_ref, b1_ref, W2_ref, b2_ref,
                Wout_ref, bout_ref, Wlin_ref, blin_ref, out_ref,
                adj_vmem, x_vmem, adj_sems, x_sems):
    bf = jnp.bfloat16

    adj_copies = []
    x_copies = []
    for i in range(B):
        ca = pltpu.make_async_copy(adj_hbm.at[i], adj_vmem.at[i],
                                   adj_sems.at[i])
        cx = pltpu.make_async_copy(x_hbm.at[i], x_vmem.at[i], x_sems.at[i])
        ca.start()
        cx.start()
        adj_copies.append(ca)
        x_copies.append(cx)

    def dot(a, b):
        return jnp.dot(a, b, preferred_element_type=jnp.float32)

    for g in range(B // G):
        docs = range(g * G, (g + 1) * G)
        for i in docs:
            adj_copies[i].wait()
            x_copies[i].wait()
        A = [adj_vmem[i].astype(bf) for i in docs]
        deg = [jnp.sum(a.astype(jnp.float32), axis=1) for a in A]
        dis = [jax.lax.rsqrt(jnp.maximum(d, 1e-12))[:, None] for d in deg]
        xb = [x_vmem[i].astype(bf) for i in docs]

        xw = [dot(xv, W1_ref[:, :]) for xv in xb]
        s = [(dis[k] * xw[k]).astype(bf) for k in range(G)]
        u = [dot(A[k], s[k]) for k in range(G)]
        g1 = [(dis[k] * jnp.maximum(dis[k] * u[k] + b1_ref[:, :], 0.0)
               ).astype(bf) for k in range(G)]
        s = [dot(g1[k], W2_ref[:, :]).astype(bf) for k in range(G)]
        u = [dot(A[k], s[k]) for k in range(G)]
        g2 = [(dis[k] * jnp.maximum(dis[k] * u[k] + b2_ref[:, :], 0.0)
               ).astype(bf) for k in range(G)]
        s = [dot(g2[k], Wout_ref[:, :]).astype(bf) for k in range(G)]
        u = [dot(A[k], s[k]) for k in range(G)]
        nv = [dis[k] * u[k] + bout_ref[:, :] for k in range(G)]

        ge = [jnp.max(v, axis=0, keepdims=True).astype(bf) for v in nv]
        for k, i in enumerate(docs):
            out_ref[i, :, :] = dot(ge[k], Wlin_ref[:, :]) + blin_ref[:, :]


@functools.partial(jax.jit, static_argnames=())
def kernel(x, init_adj, W1, b1, W2, b2, Wout, bout, W_lin, b_lin):
    bf = jnp.bfloat16
    vm = pltpu.MemorySpace.VMEM
    out = pl.pallas_call(
        _gcn_kernel,
        in_specs=[
            pl.BlockSpec(memory_space=pltpu.MemorySpace.HBM),
            pl.BlockSpec(memory_space=pltpu.MemorySpace.HBM),
            pl.BlockSpec(memory_space=vm),
            pl.BlockSpec(memory_space=vm),
            pl.BlockSpec(memory_space=vm),
            pl.BlockSpec(memory_space=vm),
            pl.BlockSpec(memory_space=vm),
            pl.BlockSpec(memory_space=vm),
            pl.BlockSpec(memory_space=vm),
            pl.BlockSpec(memory_space=vm),
        ],
        out_specs=pl.BlockSpec(memory_space=vm),
        out_shape=jax.ShapeDtypeStruct((B, 1, R), jnp.float32),
        scratch_shapes=[
            pltpu.VMEM((B, N, N), jnp.float32),
            pltpu.VMEM((B, N, F), jnp.float32),
            pltpu.SemaphoreType.DMA((B,)),
            pltpu.SemaphoreType.DMA((B,)),
        ],
        compiler_params=pltpu.CompilerParams(
            vmem_limit_bytes=100 * 1024 * 1024,
        ),
    )(x, init_adj,
      W1.astype(bf), b1.reshape(1, H),
      W2.astype(bf), b2.reshape(1, H),
      Wout.astype(bf), bout.reshape(1, O),
      W_lin.astype(bf), b_lin.reshape(1, R))
    return out.reshape(B, R)


# speedup vs baseline: 1.0109x; 1.0109x over previous
import functools

import jax
import jax.numpy as jnp
from jax.experimental import pallas as pl
from jax.experimental.pallas import tpu as pltpu

B, N, F, H, O, R = 32, 512, 256, 128, 128, 53
G = 4


def _gcn_kernel(x_hbm, adj_hbm, W1_ref, b1_ref, W2_ref, b2_ref,
                Wout_ref, bout_ref, Wlin_ref, blin_ref, out_ref,
                adj_vmem, x_vmem, adj_sems, x_sems):
    bf = jnp.bfloat16

    adj_copies = [pltpu.make_async_copy(adj_hbm.at[i], adj_vmem.at[i],
                                        adj_sems.at[i]) for i in range(B)]
    x_copies = [pltpu.make_async_copy(x_hbm.at[i], x_vmem.at[i],
                                      x_sems.at[i]) for i in range(B)]

    AHEAD = 3

    def start_group(gg):
        for i in range(gg * G, (gg + 1) * G):
            adj_copies[i].start()
            x_copies[i].start()

    for gg in range(AHEAD):
        start_group(gg)

    def dot(a, b):
        return jnp.dot(a, b, preferred_element_type=jnp.float32)

    for g in range(B // G):
        docs = range(g * G, (g + 1) * G)
        for i in docs:
            adj_copies[i].wait()
            x_copies[i].wait()
        if g + AHEAD < B // G:
            start_group(g + AHEAD)
        A = [adj_vmem[i].astype(bf) for i in docs]
        deg = [jnp.sum(a.astype(jnp.float32), axis=1) for a in A]
        dis = [jax.lax.rsqrt(jnp.maximum(d, 1e-12))[:, None] for d in deg]
        xb = [x_vmem[i].astype(bf) for i in docs]

        xw = [dot(xv, W1_ref[:, :]) for xv in xb]
        s = [(dis[k] * xw[k]).astype(bf) for k in range(G)]
        u = [dot(A[k], s[k]) for k in range(G)]
        g1 = [(dis[k] * jnp.maximum(dis[k] * u[k] + b1_ref[:, :], 0.0)
               ).astype(bf) for k in range(G)]
        s = [dot(g1[k], W2_ref[:, :]).astype(bf) for k in range(G)]
        u = [dot(A[k], s[k]) for k in range(G)]
        g2 = [(dis[k] * jnp.maximum(dis[k] * u[k] + b2_ref[:, :], 0.0)
               ).astype(bf) for k in range(G)]
        s = [dot(g2[k], Wout_ref[:, :]).astype(bf) for k in range(G)]
        u = [dot(A[k], s[k]) for k in range(G)]
        nv = [dis[k] * u[k] + bout_ref[:, :] for k in range(G)]

        ge = [jnp.max(v, axis=0, keepdims=True).astype(bf) for v in nv]
        for k, i in enumerate(docs):
            out_ref[i, :, :] = dot(ge[k], Wlin_ref[:, :]) + blin_ref[:, :]


@functools.partial(jax.jit, static_argnames=())
def kernel(x, init_adj, W1, b1, W2, b2, Wout, bout, W_lin, b_lin):
    bf = jnp.bfloat16
    vm = pltpu.MemorySpace.VMEM
    out = pl.pallas_call(
        _gcn_kernel,
        in_specs=[
            pl.BlockSpec(memory_space=pltpu.MemorySpace.HBM),
            pl.BlockSpec(memory_space=pltpu.MemorySpace.HBM),
            pl.BlockSpec(memory_space=vm),
            pl.BlockSpec(memory_space=vm),
            pl.BlockSpec(memory_space=vm),
            pl.BlockSpec(memory_space=vm),
            pl.BlockSpec(memory_space=vm),
            pl.BlockSpec(memory_space=vm),
            pl.BlockSpec(memory_space=vm),
            pl.BlockSpec(memory_space=vm),
        ],
        out_specs=pl.BlockSpec(memory_space=vm),
        out_shape=jax.ShapeDtypeStruct((B, 1, R), jnp.float32),
        scratch_shapes=[
            pltpu.VMEM((B, N, N), jnp.float32),
            pltpu.VMEM((B, N, F), jnp.float32),
            pltpu.SemaphoreType.DMA((B,)),
            pltpu.SemaphoreType.DMA((B,)),
        ],
        compiler_params=pltpu.CompilerParams(
            vmem_limit_bytes=100 * 1024 * 1024,
        ),
    )(x, init_adj,
      W1.astype(bf), b1.reshape(1, H),
      W2.astype(bf), b2.reshape(1, H),
      Wout.astype(bf), bout.reshape(1, O),
      W_lin.astype(bf), b_lin.reshape(1, R))
    return out.reshape(B, R)


# speedup vs baseline: 1.1883x; 1.1755x over previous
import functools

import jax
import jax.numpy as jnp
from jax.experimental import pallas as pl
from jax.experimental.pallas import tpu as pltpu

B, N, F, H, O, R = 32, 512, 256, 128, 128, 53
G = 4


def _gcn_kernel(x_hbm, adj_hbm, W1_ref, b1_ref, W2_ref, b2_ref,
                Wout_ref, bout_ref, Wlin_ref, blin_ref, out_ref,
                adj_vmem, x_vmem, adj_sems, x_sems):
    bf = jnp.bfloat16

    adj_copies = [pltpu.make_async_copy(adj_hbm.at[i], adj_vmem.at[i],
                                        adj_sems.at[i]) for i in range(B)]
    x_copies = [pltpu.make_async_copy(x_hbm.at[i], x_vmem.at[i],
                                      x_sems.at[i]) for i in range(B)]

    AHEAD = 3

    def start_group(gg):
        for i in range(gg * G, (gg + 1) * G):
            adj_copies[i].start()
            x_copies[i].start()

    for gg in range(AHEAD):
        start_group(gg)

    def dot(a, b):
        return jnp.dot(a, b, preferred_element_type=jnp.float32)

    for g in range(B // G):
        docs = range(g * G, (g + 1) * G)
        for i in docs:
            adj_copies[i].wait()
            x_copies[i].wait()
        if g + AHEAD < B // G:
            start_group(g + AHEAD)
        A = [adj_vmem[i] for i in docs]
        deg = [jnp.sum(a, axis=1) for a in A]
        dis = [jax.lax.rsqrt(jnp.maximum(d, 1e-12))[:, None] for d in deg]
        xb = [x_vmem[i] for i in docs]

        xw = [dot(xv, W1_ref[:, :]) for xv in xb]
        s = [dis[k] * xw[k] for k in range(G)]
        u = [dot(A[k], s[k]) for k in range(G)]
        g1 = [dis[k] * jnp.maximum(dis[k] * u[k] + b1_ref[:, :], 0.0)
              for k in range(G)]
        s = [dot(g1[k], W2_ref[:, :]) for k in range(G)]
        u = [dot(A[k], s[k]) for k in range(G)]
        g2 = [dis[k] * jnp.maximum(dis[k] * u[k] + b2_ref[:, :], 0.0)
              for k in range(G)]
        s = [dot(g2[k], Wout_ref[:, :]) for k in range(G)]
        u = [dot(A[k], s[k]) for k in range(G)]
        nv = [dis[k] * u[k] + bout_ref[:, :] for k in range(G)]

        ge = [jnp.max(v, axis=0, keepdims=True) for v in nv]
        for k, i in enumerate(docs):
            out_ref[i, :, :] = dot(ge[k], Wlin_ref[:, :]) + blin_ref[:, :]


@functools.partial(jax.jit, static_argnames=())
def kernel(x, init_adj, W1, b1, W2, b2, Wout, bout, W_lin, b_lin):
    bf = jnp.bfloat16
    vm = pltpu.MemorySpace.VMEM
    out = pl.pallas_call(
        _gcn_kernel,
        in_specs=[
            pl.BlockSpec(memory_space=pltpu.MemorySpace.HBM),
            pl.BlockSpec(memory_space=pltpu.MemorySpace.HBM),
            pl.BlockSpec(memory_space=vm),
            pl.BlockSpec(memory_space=vm),
            pl.BlockSpec(memory_space=vm),
            pl.BlockSpec(memory_space=vm),
            pl.BlockSpec(memory_space=vm),
            pl.BlockSpec(memory_space=vm),
            pl.BlockSpec(memory_space=vm),
            pl.BlockSpec(memory_space=vm),
        ],
        out_specs=pl.BlockSpec(memory_space=vm),
        out_shape=jax.ShapeDtypeStruct((B, 1, R), jnp.float32),
        scratch_shapes=[
            pltpu.VMEM((B, N, N), jnp.float32),
            pltpu.VMEM((B, N, F), jnp.float32),
            pltpu.SemaphoreType.DMA((B,)),
            pltpu.SemaphoreType.DMA((B,)),
        ],
        compiler_params=pltpu.CompilerParams(
            vmem_limit_bytes=100 * 1024 * 1024,
        ),
    )(x, init_adj,
      W1, b1.reshape(1, H),
      W2, b2.reshape(1, H),
      Wout, bout.reshape(1, O),
      W_lin, b_lin.reshape(1, R))
    return out.reshape(B, R)


# speedup vs baseline: 1.2340x; 1.0385x over previous
import functools

import jax
import jax.numpy as jnp
from jax.experimental import pallas as pl
from jax.experimental.pallas import tpu as pltpu

B, N, F, H, O, R = 32, 512, 256, 128, 128, 53
G = 4


def _gcn_kernel(x_hbm, adj_hbm, W1_ref, b1_ref, W2_ref, b2_ref,
                Wout_ref, bout_ref, Wlin_ref, blin_ref, out_ref,
                adj_vmem, x_vmem, ge_vmem, adj_sems, x_sems):
    bf = jnp.bfloat16

    adj_copies = [pltpu.make_async_copy(adj_hbm.at[i], adj_vmem.at[i],
                                        adj_sems.at[i]) for i in range(B)]
    x_copies = [pltpu.make_async_copy(x_hbm.at[i], x_vmem.at[i],
                                      x_sems.at[i]) for i in range(B)]

    AHEAD = 3

    def start_group(gg):
        for i in range(gg * G, (gg + 1) * G):
            adj_copies[i].start()
            x_copies[i].start()

    for gg in range(AHEAD):
        start_group(gg)

    def dot(a, b):
        return jnp.dot(a, b, preferred_element_type=jnp.float32)

    for g in range(B // G):
        docs = range(g * G, (g + 1) * G)
        for i in docs:
            adj_copies[i].wait()
            x_copies[i].wait()
        if g + AHEAD < B // G:
            start_group(g + AHEAD)
        A = [adj_vmem[i] for i in docs]
        deg = [jnp.sum(a, axis=1) for a in A]
        dis = [jax.lax.rsqrt(jnp.maximum(d, 1e-12))[:, None] for d in deg]
        xb = [x_vmem[i] for i in docs]

        xw = [dot(xv, W1_ref[:, :]) for xv in xb]
        s = [dis[k] * xw[k] for k in range(G)]
        u = [dot(A[k], s[k]) for k in range(G)]
        g1 = [dis[k] * jnp.maximum(dis[k] * u[k] + b1_ref[:, :], 0.0)
              for k in range(G)]
        s = [dot(g1[k], W2_ref[:, :]) for k in range(G)]
        u = [dot(A[k], s[k]) for k in range(G)]
        g2 = [dis[k] * jnp.maximum(dis[k] * u[k] + b2_ref[:, :], 0.0)
              for k in range(G)]
        s = [dot(g2[k], Wout_ref[:, :]) for k in range(G)]
        u = [dot(A[k], s[k]) for k in range(G)]
        nv = [dis[k] * u[k] + bout_ref[:, :] for k in range(G)]

        ge = [jnp.max(v, axis=0, keepdims=True) for v in nv]
        for k, i in enumerate(docs):
            ge_vmem[i, :] = ge[k][0]

    out_ref[:, 0, :] = dot(ge_vmem[:, :], Wlin_ref[:, :]) + blin_ref[:, :]


@functools.partial(jax.jit, static_argnames=())
def kernel(x, init_adj, W1, b1, W2, b2, Wout, bout, W_lin, b_lin):
    bf = jnp.bfloat16
    vm = pltpu.MemorySpace.VMEM
    out = pl.pallas_call(
        _gcn_kernel,
        in_specs=[
            pl.BlockSpec(memory_space=pltpu.MemorySpace.HBM),
            pl.BlockSpec(memory_space=pltpu.MemorySpace.HBM),
            pl.BlockSpec(memory_space=vm),
            pl.BlockSpec(memory_space=vm),
            pl.BlockSpec(memory_space=vm),
            pl.BlockSpec(memory_space=vm),
            pl.BlockSpec(memory_space=vm),
            pl.BlockSpec(memory_space=vm),
            pl.BlockSpec(memory_space=vm),
            pl.BlockSpec(memory_space=vm),
        ],
        out_specs=pl.BlockSpec(memory_space=vm),
        out_shape=jax.ShapeDtypeStruct((B, 1, R), jnp.float32),
        scratch_shapes=[
            pltpu.VMEM((B, N, N), jnp.float32),
            pltpu.VMEM((B, N, F), jnp.float32),
            pltpu.VMEM((B, O), jnp.float32),
            pltpu.SemaphoreType.DMA((B,)),
            pltpu.SemaphoreType.DMA((B,)),
        ],
        compiler_params=pltpu.CompilerParams(
            vmem_limit_bytes=100 * 1024 * 1024,
        ),
    )(x, init_adj,
      W1, b1.reshape(1, H),
      W2, b2.reshape(1, H),
      Wout, bout.reshape(1, O),
      W_lin, b_lin.reshape(1, R))
    return out.reshape(B, R)
